# Initial kernel scaffold; baseline (speedup 1.0000x reference)
import jax
import jax.numpy as jnp
from jax.experimental import pallas as pl


def kernel(fmap1, fmap2, fmap3, depths):
    raise NotImplementedError("write your pallas kernel here")


# speedup vs baseline: 56.4793x; 56.4793x over previous
import numpy as np
import jax
import jax.numpy as jnp
from jax import lax
from jax.experimental import pallas as pl
from jax.experimental.pallas import tpu as pltpu

_D = 10
_HW = 224
_LAYERS = ((96, 56), (192, 28), (384, 14))


def _wmat(in_size):
    c = (np.arange(_HW) + 0.5) * in_size / _HW - 0.5
    w = np.maximum(0.0, 1.0 - np.abs(c[:, None] - np.arange(in_size)[None, :]))
    return (w / w.sum(1, keepdims=True)).astype(np.float32)


_WX = {h: _wmat(h) for _, h in _LAYERS}
_WYT = {h: _wmat(h).T.copy() for _, h in _LAYERS}
_SMAT = np.zeros((_D, _D * _HW), np.float32)
for _d in range(_D):
    _SMAT[_d, _d * _HW:(_d + 1) * _HW] = 1.0


def _c1_body(dref, sref, wx1, wyt1, wx2, wyt2, wx3, wyt3,
             o1, o2, o3, cref):
    d3 = dref[...]
    mn = jnp.min(d3)
    mx = jnp.max(d3)
    step = (mx - mn) / np.float32(_D)
    bi = jnp.zeros(d3.shape, jnp.int32)
    for k in range(1, _D):
        bi = bi + (d3 >= mn + step * np.float32(k)).astype(jnp.int32)
    cnt = jnp.zeros((_D, 1), jnp.float32)
    for b in range(2):
        bib = bi[b]
        tall = jnp.concatenate(
            [(bib == dd).astype(jnp.float32) for dd in range(_D)], axis=0)
        cnt = cnt + jnp.dot(sref[...], jnp.sum(tall, axis=1, keepdims=True))
        for wxr, wytr, oref in ((wx1, wyt1, o1), (wx2, wyt2, o2),
                                (wx3, wyt3, o3)):
            t1 = jnp.dot(tall, wxr[...])
            wide = jnp.concatenate(
                [t1[dd * _HW:(dd + 1) * _HW, :] for dd in range(_D)], axis=1)
            oref[b] = jnp.dot(wytr[...], wide)
    cref[...] = cnt


def _c2_body(a1, b1, a2, b2, a3, b3, cref, out):
    inv = 1.0 / jnp.maximum(cref[...], np.float32(1e-6))
    out[...] = jnp.zeros((3, 384, _D), jnp.float32)
    out[0, 0:96, :] = jnp.dot(a1[...], b1[...]) * inv
    out[1, 0:192, :] = jnp.dot(a2[...], b2[...]) * inv
    out[2, :, :] = jnp.dot(a3[...], b3[...]) * inv


def kernel(fmap1, fmap2, fmap3, depths):
    d3 = depths.reshape(2, _HW, _HW)
    c1_out = pl.pallas_call(
        _c1_body,
        out_shape=[
            jax.ShapeDtypeStruct((2, 56, _D * 56), jnp.float32),
            jax.ShapeDtypeStruct((2, 28, _D * 28), jnp.float32),
            jax.ShapeDtypeStruct((2, 14, _D * 14), jnp.float32),
            jax.ShapeDtypeStruct((_D, 1), jnp.float32),
        ],
    )(d3, jnp.asarray(_SMAT),
      jnp.asarray(_WX[56]), jnp.asarray(_WYT[56]),
      jnp.asarray(_WX[28]), jnp.asarray(_WYT[28]),
      jnp.asarray(_WX[14]), jnp.asarray(_WYT[14]))
    o_by_layer = c1_out[:3]
    cnt = c1_out[3].reshape(1, _D)

    mats = []
    for (c, h), o in zip(_LAYERS, o_by_layer):
        f = (fmap1, fmap2, fmap3)[len(mats) // 2]
        a = f.transpose(1, 0, 2, 3).reshape(c, 2 * h * h)
        bm = o.reshape(2, h, _D, h).transpose(0, 1, 3, 2).reshape(2 * h * h, _D)
        mats += [a, bm]

    return pl.pallas_call(
        _c2_body,
        out_shape=jax.ShapeDtypeStruct((3, 384, _D), jnp.float32),
    )(*mats, cnt)
